# Initial kernel scaffold; baseline (speedup 1.0000x reference)
import jax
import jax.numpy as jnp
from jax.experimental import pallas as pl


def kernel(logits, logits_teacher, targets):
    raise NotImplementedError("write your pallas kernel here")


# speedup vs baseline: 8.3913x; 8.3913x over previous
import functools

import jax
import jax.numpy as jnp
import numpy as np
from jax.experimental import pallas as pl
from jax.experimental.pallas import tpu as pltpu

_GROUP_SIZES = (21, 75, 150, 304, 700, 1500, 3000, 4700)
_NUM_CLASSES = int(np.sum(_GROUP_SIZES))
_KMAX = 500
_B = 1024
_RB = 128
_NEG_INF = float("-inf")


def _group_windows():
    offs = np.cumsum([0] + list(_GROUP_SIZES))
    wins = []
    for g, size in enumerate(_GROUP_SIZES):
        off, end = int(offs[g]), int(offs[g + 1])
        ws = (off // 128) * 128
        we = min(((end + 127) // 128) * 128, _NUM_CLASSES)
        wins.append((off, end, ws, we, min(size, _KMAX)))
    return wins


_WINDOWS = _group_windows()
_I32_MIN = np.int32(np.iinfo(np.int32).min)
_I32_MAX = np.int32(np.iinfo(np.int32).max)


def _kth_largest_key(key, k):
    rows = key.shape[0]
    lo = jnp.full((rows, 1), _I32_MIN, dtype=jnp.int32)
    hi = jnp.full((rows, 1), _I32_MAX, dtype=jnp.int32)

    def body(_, carry):
        lo, hi = carry
        mid = (lo >> 1) + (hi >> 1) + ((lo | hi) & 1)
        cnt = jnp.sum((key >= mid).astype(jnp.int32), axis=1, keepdims=True)
        ge = cnt >= k
        return jnp.where(ge, mid, lo), jnp.where(ge, hi, mid - 1)

    lo, hi = jax.lax.fori_loop(0, 32, body, (lo, hi))
    return lo


def _loss_body(s_ref, t_ref, o_ref):
    pid = pl.program_id(0)
    total = jnp.float32(0.0)
    for g, (off, end, ws, we, k) in enumerate(_WINDOWS):
        size = end - off
        t = t_ref[:, ws:we]
        s = s_ref[:, ws:we]
        cols = jax.lax.broadcasted_iota(jnp.int32, t.shape, 1) + ws
        mask = (cols >= off) & (cols < end)

        if k == size:
            sel = mask
            wsel = mask.astype(jnp.float32)
        else:
            ti = jax.lax.bitcast_convert_type(t, jnp.int32)
            key = jnp.where(ti < 0, ti ^ jnp.int32(0x7FFFFFFF), ti)
            key = jnp.where(mask, key, _I32_MIN)
            tau = _kth_largest_key(key, k)
            gt = key > tau
            eq = key == tau
            cgt = jnp.sum(gt.astype(jnp.float32), axis=1, keepdims=True)
            ceq = jnp.sum(eq.astype(jnp.float32), axis=1, keepdims=True)
            frac = (jnp.float32(k) - cgt) / ceq
            wsel = jnp.where(gt, jnp.float32(1.0), jnp.where(eq, frac, 0.0))
            sel = gt | eq

        t_eff = jnp.where(sel, t, _NEG_INF)
        m_t = jnp.max(t_eff, axis=1, keepdims=True)
        w = wsel * jnp.exp(t_eff - m_t)
        z_t = jnp.sum(w, axis=1, keepdims=True)
        s_tt = jnp.sum(w * jnp.where(sel, t - m_t, 0.0), axis=1, keepdims=True)
        s_ts = jnp.sum(w * jnp.where(sel, s, 0.0), axis=1, keepdims=True)
        s_eff = jnp.where(sel, s, _NEG_INF)
        m_s = jnp.max(s_eff, axis=1, keepdims=True)
        z_s = jnp.sum(wsel * jnp.exp(s_eff - m_s), axis=1, keepdims=True)
        kl = (s_tt - s_ts) / z_t - jnp.log(z_t) + m_s + jnp.log(z_s)
        norm = jnp.float32(size / float(_NUM_CLASSES) / float(_B))
        total = total + jnp.sum(kl) * norm

    o_ref[0, 0] = jnp.where(pid == 0, total, o_ref[0, 0] + total)


@jax.jit
def kernel(logits, logits_teacher, targets):
    del targets
    out = pl.pallas_call(
        _loss_body,
        grid=(_B // _RB,),
        in_specs=[
            pl.BlockSpec((_RB, _NUM_CLASSES), lambda i: (i, 0)),
            pl.BlockSpec((_RB, _NUM_CLASSES), lambda i: (i, 0)),
        ],
        out_specs=pl.BlockSpec(memory_space=pltpu.SMEM),
        out_shape=jax.ShapeDtypeStruct((1, 1), jnp.float32),
    )(logits, logits_teacher)
    return out[0, 0]


# speedup vs baseline: 8.9175x; 1.0627x over previous
import functools

import jax
import jax.numpy as jnp
import numpy as np
from jax.experimental import pallas as pl
from jax.experimental.pallas import tpu as pltpu

_GROUP_SIZES = (21, 75, 150, 304, 700, 1500, 3000, 4700)
_NUM_CLASSES = int(np.sum(_GROUP_SIZES))
_KMAX = 500
_B = 1024
_RB = 128
_NEG_INF = float("-inf")
_LO_INIT = np.int32(-2139095042)
_HI_INIT = np.int32(2139095040)


def _group_windows():
    offs = np.cumsum([0] + list(_GROUP_SIZES))
    wins = []
    for g, size in enumerate(_GROUP_SIZES):
        off, end = int(offs[g]), int(offs[g + 1])
        ws = (off // 128) * 128
        we = min(((end + 127) // 128) * 128, _NUM_CLASSES)
        wins.append((off, end, ws, we, min(size, _KMAX)))
    return wins


_WINDOWS = _group_windows()


def _key_to_f32(m):
    ti = jnp.where(m < 0, m ^ jnp.int32(0x7FFFFFFF), m)
    return jax.lax.bitcast_convert_type(ti, jnp.float32)


def _reduce_terms(sel, wsel, t, s, tm, rows_norm):
    t_eff = jnp.where(sel, tm, _NEG_INF)
    m_t = jnp.max(t_eff, axis=1, keepdims=True)
    w = wsel * jnp.exp(t_eff - m_t)
    z_t = jnp.sum(w, axis=1, keepdims=True)
    s_wt = jnp.sum(w * t, axis=1, keepdims=True)
    s_ts = jnp.sum(w * s, axis=1, keepdims=True)
    s_eff = jnp.where(sel, s, _NEG_INF)
    m_s = jnp.max(s_eff, axis=1, keepdims=True)
    z_s = jnp.sum(wsel * jnp.exp(s_eff - m_s), axis=1, keepdims=True)
    kl = (s_wt - m_t * z_t - s_ts) / z_t - jnp.log(z_t) + m_s + jnp.log(z_s)
    return jnp.sum(kl) * rows_norm


def _loss_body(s_ref, t_ref, o_ref):
    pid = pl.program_id(0)
    total = jnp.float32(0.0)
    big = []
    for g, (off, end, ws, we, k) in enumerate(_WINDOWS):
        size = end - off
        t = t_ref[:, ws:we]
        s = s_ref[:, ws:we]
        cols = jax.lax.broadcasted_iota(jnp.int32, t.shape, 1) + ws
        mask = (cols >= off) & (cols < end)
        tm = jnp.where(mask, t, _NEG_INF)
        norm = jnp.float32(size / float(_NUM_CLASSES) / float(_B))
        if k == size:
            total = total + _reduce_terms(
                mask, mask.astype(jnp.float32), t, s, tm, norm)
        else:
            big.append((t, s, tm, k, norm))

    nbig = len(big)
    ones = [jnp.ones((b[0].shape[1], 1), jnp.float32) for b in big]
    rows = big[0][0].shape[0]
    los = tuple(jnp.full((rows, 1), _LO_INIT, jnp.int32) for _ in range(nbig))
    his = tuple(jnp.full((rows, 1), _HI_INIT, jnp.int32) for _ in range(nbig))

    def body(_, carry):
        los, his = carry
        nlos, nhis = [], []
        for gi in range(nbig):
            lo, hi = los[gi], his[gi]
            mid = (lo >> 1) + (hi >> 1) + ((lo | hi) & 1)
            f_mid = _key_to_f32(mid)
            ind = jnp.where(big[gi][2] >= f_mid, 1.0, 0.0).astype(jnp.float32)
            cnt = jax.lax.dot_general(
                ind, ones[gi], (((1,), (0,)), ((), ())),
                preferred_element_type=jnp.float32)
            ge = cnt >= jnp.float32(big[gi][3])
            nlos.append(jnp.where(ge, mid, lo))
            nhis.append(jnp.where(ge, hi, mid - 1))
        return tuple(nlos), tuple(nhis)

    los, his = jax.lax.fori_loop(0, 32, body, (los, his))

    for gi in range(nbig):
        t, s, tm, k, norm = big[gi]
        f_tau = _key_to_f32(los[gi])
        sel = tm >= f_tau
        gt = tm > f_tau
        eq = jnp.logical_xor(sel, gt)
        cgt = jnp.sum(gt.astype(jnp.float32), axis=1, keepdims=True)
        ceq = jnp.sum(eq.astype(jnp.float32), axis=1, keepdims=True)
        frac = (jnp.float32(k) - cgt) / ceq
        wsel = jnp.where(gt, jnp.float32(1.0), jnp.where(eq, frac, 0.0))
        total = total + _reduce_terms(sel, wsel, t, s, tm, norm)

    o_ref[0, 0] = jnp.where(pid == 0, total, o_ref[0, 0] + total)


@jax.jit
def kernel(logits, logits_teacher, targets):
    del targets
    out = pl.pallas_call(
        _loss_body,
        grid=(_B // _RB,),
        in_specs=[
            pl.BlockSpec((_RB, _NUM_CLASSES), lambda i: (i, 0)),
            pl.BlockSpec((_RB, _NUM_CLASSES), lambda i: (i, 0)),
        ],
        out_specs=pl.BlockSpec(memory_space=pltpu.SMEM),
        out_shape=jax.ShapeDtypeStruct((1, 1), jnp.float32),
    )(logits, logits_teacher)
    return out[0, 0]


# speedup vs baseline: 8.9280x; 1.0012x over previous
import functools

import jax
import jax.numpy as jnp
import numpy as np
from jax.experimental import pallas as pl
from jax.experimental.pallas import tpu as pltpu

_GROUP_SIZES = (21, 75, 150, 304, 700, 1500, 3000, 4700)
_NUM_CLASSES = int(np.sum(_GROUP_SIZES))
_KMAX = 500
_B = 1024
_RB = 128
_NEG_INF = float("-inf")
_LO_INIT = np.int32(-2139095042)
_HI_INIT = np.int32(2139095040)


def _group_windows():
    offs = np.cumsum([0] + list(_GROUP_SIZES))
    wins = []
    for g, size in enumerate(_GROUP_SIZES):
        off, end = int(offs[g]), int(offs[g + 1])
        ws = (off // 128) * 128
        we = min(((end + 127) // 128) * 128, _NUM_CLASSES)
        wins.append((off, end, ws, we, min(size, _KMAX)))
    return wins


_WINDOWS = _group_windows()


def _key_to_f32(m):
    ti = jnp.where(m < 0, m ^ jnp.int32(0x7FFFFFFF), m)
    return jax.lax.bitcast_convert_type(ti, jnp.float32)


def _rowsum(x, ones):
    return jax.lax.dot_general(
        x, ones, (((1,), (0,)), ((), ())), preferred_element_type=jnp.float32)


def _kl_terms(wsel, e_t, e_s, t, s, m_t, m_s, ones, rows_norm):
    w = wsel * e_t
    z_t = _rowsum(w, ones)
    s_wt = _rowsum(w * t, ones)
    s_ts = _rowsum(w * s, ones)
    z_s = _rowsum(wsel * e_s, ones)
    kl = (s_wt - m_t * z_t - s_ts) / z_t - jnp.log(z_t) + m_s + jnp.log(z_s)
    return jnp.sum(kl) * rows_norm


def _loss_body(s_ref, t_ref, o_ref):
    pid = pl.program_id(0)
    total = jnp.float32(0.0)
    big = []
    for g, (off, end, ws, we, k) in enumerate(_WINDOWS):
        size = end - off
        t = t_ref[:, ws:we]
        s = s_ref[:, ws:we]
        cols = jax.lax.broadcasted_iota(jnp.int32, t.shape, 1) + ws
        mask = (cols >= off) & (cols < end)
        tm = jnp.where(mask, t, _NEG_INF)
        sm = jnp.where(mask, s, _NEG_INF)
        ones = jnp.ones((t.shape[1], 1), jnp.float32)
        norm = jnp.float32(size / float(_NUM_CLASSES) / float(_B))
        if k == size:
            m_t = jnp.max(tm, axis=1, keepdims=True)
            m_s = jnp.max(sm, axis=1, keepdims=True)
            e_t = jnp.exp(tm - m_t)
            e_s = jnp.exp(sm - m_s)
            total = total + _kl_terms(
                jnp.float32(1.0), e_t, e_s, t, s, m_t, m_s, ones, norm)
        else:
            big.append((t, s, tm, sm, k, norm, ones))

    nbig = len(big)
    rows = big[0][0].shape[0]
    los = tuple(jnp.full((rows, 1), _LO_INIT, jnp.int32) for _ in range(nbig))
    his = tuple(jnp.full((rows, 1), _HI_INIT, jnp.int32) for _ in range(nbig))

    def body(_, carry):
        los, his = carry
        nlos, nhis = [], []
        for gi in range(nbig):
            lo, hi = los[gi], his[gi]
            mid = (lo >> 1) + (hi >> 1) + ((lo | hi) & 1)
            f_mid = _key_to_f32(mid)
            ind = jnp.where(big[gi][2] >= f_mid, 1.0, 0.0)
            cnt = _rowsum(ind, big[gi][6])
            ge = cnt >= jnp.float32(big[gi][4])
            nlos.append(jnp.where(ge, mid, lo))
            nhis.append(jnp.where(ge, hi, mid - 1))
        return tuple(nlos), tuple(nhis)

    los, his = jax.lax.fori_loop(0, 32, body, (los, his))

    for gi in range(nbig):
        t, s, tm, sm, k, norm, ones = big[gi]
        f_tau = _key_to_f32(los[gi])
        gt01 = jnp.where(tm > f_tau, 1.0, 0.0)
        eq01 = jnp.where(tm == f_tau, 1.0, 0.0)
        cgt = _rowsum(gt01, ones)
        ceq = _rowsum(eq01, ones)
        frac = (jnp.float32(k) - cgt) / ceq
        wsel = gt01 + frac * eq01
        m_t = jnp.max(tm, axis=1, keepdims=True)
        m_s = jnp.max(sm, axis=1, keepdims=True)
        e_t = jnp.exp(tm - m_t)
        e_s = jnp.exp(sm - m_s)
        total = total + _kl_terms(wsel, e_t, e_s, t, s, m_t, m_s, ones, norm)

    o_ref[0, 0] = jnp.where(pid == 0, total, o_ref[0, 0] + total)


@jax.jit
def kernel(logits, logits_teacher, targets):
    del targets
    out = pl.pallas_call(
        _loss_body,
        grid=(_B // _RB,),
        in_specs=[
            pl.BlockSpec((_RB, _NUM_CLASSES), lambda i: (i, 0)),
            pl.BlockSpec((_RB, _NUM_CLASSES), lambda i: (i, 0)),
        ],
        out_specs=pl.BlockSpec(memory_space=pltpu.SMEM),
        out_shape=jax.ShapeDtypeStruct((1, 1), jnp.float32),
    )(logits, logits_teacher)
    return out[0, 0]


# speedup vs baseline: 10.1955x; 1.1420x over previous
import functools

import jax
import jax.numpy as jnp
import numpy as np
from jax.experimental import pallas as pl
from jax.experimental.pallas import tpu as pltpu

_GROUP_SIZES = (21, 75, 150, 304, 700, 1500, 3000, 4700)
_NUM_CLASSES = int(np.sum(_GROUP_SIZES))
_KMAX = 500
_B = 1024
_RB = 128
_NEG_INF = float("-inf")
_LO_INIT = np.int32(-2139095042)
_HI_INIT = np.int32(2139095040)


def _group_windows():
    offs = np.cumsum([0] + list(_GROUP_SIZES))
    wins = []
    for g, size in enumerate(_GROUP_SIZES):
        off, end = int(offs[g]), int(offs[g + 1])
        ws = (off // 128) * 128
        we = min(((end + 127) // 128) * 128, _NUM_CLASSES)
        wins.append((off, end, ws, we, min(size, _KMAX)))
    return wins


_WINDOWS = _group_windows()


def _key_to_f32(m):
    ti = jnp.where(m < 0, m ^ jnp.int32(0x7FFFFFFF), m)
    return jax.lax.bitcast_convert_type(ti, jnp.float32)


def _rowsum(x, ones):
    return jax.lax.dot_general(
        x, ones, (((1,), (0,)), ((), ())), preferred_element_type=jnp.float32)


def _kl_terms(wsel, e_t, e_s, t, s, m_t, m_s, ones, rows_norm):
    w = wsel * e_t
    z_t = _rowsum(w, ones)
    s_wt = _rowsum(w * t, ones)
    s_ts = _rowsum(w * s, ones)
    z_s = _rowsum(wsel * e_s, ones)
    kl = (s_wt - m_t * z_t - s_ts) / z_t - jnp.log(z_t) + m_s + jnp.log(z_s)
    return jnp.sum(kl) * rows_norm


def _loss_body(s_ref, t_ref, o_ref):
    pid = pl.program_id(0)
    total = jnp.float32(0.0)
    big = []
    for g, (off, end, ws, we, k) in enumerate(_WINDOWS):
        size = end - off
        t = t_ref[:, ws:we]
        s = s_ref[:, ws:we]
        cols = jax.lax.broadcasted_iota(jnp.int32, t.shape, 1) + ws
        mask = (cols >= off) & (cols < end)
        tm = jnp.where(mask, t, _NEG_INF)
        sm = jnp.where(mask, s, _NEG_INF)
        ones = jnp.ones((t.shape[1], 1), jnp.float32)
        norm = jnp.float32(size / float(_NUM_CLASSES) / float(_B))
        if k == size:
            m_t = jnp.max(tm, axis=1, keepdims=True)
            m_s = jnp.max(sm, axis=1, keepdims=True)
            e_t = jnp.exp(tm - m_t)
            e_s = jnp.exp(sm - m_s)
            total = total + _kl_terms(
                jnp.float32(1.0), e_t, e_s, t, s, m_t, m_s, ones, norm)
        else:
            big.append((t, s, tm, sm, k, norm, ones))

    nbig = len(big)
    rows = big[0][0].shape[0]
    los = tuple(jnp.full((rows, 1), _LO_INIT, jnp.int32) for _ in range(nbig))
    his = tuple(jnp.full((rows, 1), _HI_INIT, jnp.int32) for _ in range(nbig))

    def body(_, carry):
        los, his = carry
        nlos, nhis = [], []
        for gi in range(nbig):
            lo, hi = los[gi], his[gi]
            mid = (lo >> 1) + (hi >> 1) + ((lo | hi) & 1)
            f_mid = _key_to_f32(mid)
            ind = jnp.where(big[gi][2] >= f_mid, 1.0, 0.0)
            cnt = _rowsum(ind, big[gi][6])
            ge = cnt >= jnp.float32(big[gi][4])
            nlos.append(jnp.where(ge, mid, lo))
            nhis.append(jnp.where(ge, hi, mid - 1))
        return tuple(nlos), tuple(nhis)

    los, his = jax.lax.fori_loop(0, 32, body, (los, his), unroll=4)

    for gi in range(nbig):
        t, s, tm, sm, k, norm, ones = big[gi]
        f_tau = _key_to_f32(los[gi])
        gt01 = jnp.where(tm > f_tau, 1.0, 0.0)
        eq01 = jnp.where(tm == f_tau, 1.0, 0.0)
        cgt = _rowsum(gt01, ones)
        ceq = _rowsum(eq01, ones)
        frac = (jnp.float32(k) - cgt) / ceq
        wsel = gt01 + frac * eq01
        m_t = jnp.max(tm, axis=1, keepdims=True)
        m_s = jnp.max(sm, axis=1, keepdims=True)
        e_t = jnp.exp(tm - m_t)
        e_s = jnp.exp(sm - m_s)
        total = total + _kl_terms(wsel, e_t, e_s, t, s, m_t, m_s, ones, norm)

    o_ref[0, 0] = jnp.where(pid == 0, total, o_ref[0, 0] + total)


@jax.jit
def kernel(logits, logits_teacher, targets):
    del targets
    out = pl.pallas_call(
        _loss_body,
        grid=(_B // _RB,),
        in_specs=[
            pl.BlockSpec((_RB, _NUM_CLASSES), lambda i: (i, 0)),
            pl.BlockSpec((_RB, _NUM_CLASSES), lambda i: (i, 0)),
        ],
        out_specs=pl.BlockSpec(memory_space=pltpu.SMEM),
        out_shape=jax.ShapeDtypeStruct((1, 1), jnp.float32),
    )(logits, logits_teacher)
    return out[0, 0]


# speedup vs baseline: 10.4529x; 1.0252x over previous
import functools

import jax
import jax.numpy as jnp
import numpy as np
from jax.experimental import pallas as pl
from jax.experimental.pallas import tpu as pltpu

_GROUP_SIZES = (21, 75, 150, 304, 700, 1500, 3000, 4700)
_NUM_CLASSES = int(np.sum(_GROUP_SIZES))
_KMAX = 500
_B = 1024
_RB = 128
_NEG_INF = float("-inf")
_LO_INIT = np.int32(-2139095042)
_HI_INIT = np.int32(2139095040)


def _group_windows():
    offs = np.cumsum([0] + list(_GROUP_SIZES))
    wins = []
    for g, size in enumerate(_GROUP_SIZES):
        off, end = int(offs[g]), int(offs[g + 1])
        ws = (off // 128) * 128
        we = min(((end + 127) // 128) * 128, _NUM_CLASSES)
        wins.append((off, end, ws, we, min(size, _KMAX)))
    return wins


_WINDOWS = _group_windows()


def _key_to_f32(m):
    ti = jnp.where(m < 0, m ^ jnp.int32(0x7FFFFFFF), m)
    return jax.lax.bitcast_convert_type(ti, jnp.float32)


def _rowsum(x, ones):
    return jax.lax.dot_general(
        x, ones, (((1,), (0,)), ((), ())), preferred_element_type=jnp.float32)


def _kl_terms(wsel, e_t, e_s, t, s, m_t, m_s, ones, rows_norm):
    w = wsel * e_t
    z_t = _rowsum(w, ones)
    s_wt = _rowsum(w * t, ones)
    s_ts = _rowsum(w * s, ones)
    z_s = _rowsum(wsel * e_s, ones)
    kl = (s_wt - m_t * z_t - s_ts) / z_t - jnp.log(z_t) + m_s + jnp.log(z_s)
    return jnp.sum(kl) * rows_norm


def _loss_body(s_ref, t_ref, o_ref):
    pid = pl.program_id(0)
    total = jnp.float32(0.0)
    big = []
    for g, (off, end, ws, we, k) in enumerate(_WINDOWS):
        size = end - off
        t = t_ref[:, ws:we]
        s = s_ref[:, ws:we]
        cols = jax.lax.broadcasted_iota(jnp.int32, t.shape, 1) + ws
        mask = (cols >= off) & (cols < end)
        tm = jnp.where(mask, t, _NEG_INF)
        sm = jnp.where(mask, s, _NEG_INF)
        ones = jnp.ones((t.shape[1], 1), jnp.float32)
        norm = jnp.float32(size / float(_NUM_CLASSES) / float(_B))
        if k == size:
            m_t = jnp.max(tm, axis=1, keepdims=True)
            m_s = jnp.max(sm, axis=1, keepdims=True)
            e_t = jnp.exp(tm - m_t)
            e_s = jnp.exp(sm - m_s)
            total = total + _kl_terms(
                jnp.float32(1.0), e_t, e_s, t, s, m_t, m_s, ones, norm)
        else:
            big.append((t, s, tm, sm, k, norm, ones))

    nbig = len(big)
    rows = big[0][0].shape[0]
    los = tuple(jnp.full((rows, 1), _LO_INIT, jnp.int32) for _ in range(nbig))
    his = tuple(jnp.full((rows, 1), _HI_INIT, jnp.int32) for _ in range(nbig))

    def body(_, carry):
        los, his = carry
        nlos, nhis = [], []
        for gi in range(nbig):
            lo, hi = los[gi], his[gi]
            mid = (lo >> 1) + (hi >> 1) + ((lo | hi) & 1)
            f_mid = _key_to_f32(mid)
            ind = jnp.where(big[gi][2] >= f_mid, 1.0, 0.0)
            cnt = _rowsum(ind, big[gi][6])
            ge = cnt >= jnp.float32(big[gi][4])
            nlos.append(jnp.where(ge, mid, lo))
            nhis.append(jnp.where(ge, hi, mid - 1))
        return tuple(nlos), tuple(nhis)

    los, his = jax.lax.fori_loop(0, 32, body, (los, his), unroll=8)

    for gi in range(nbig):
        t, s, tm, sm, k, norm, ones = big[gi]
        f_tau = _key_to_f32(los[gi])
        gt01 = jnp.where(tm > f_tau, 1.0, 0.0)
        eq01 = jnp.where(tm == f_tau, 1.0, 0.0)
        cgt = _rowsum(gt01, ones)
        ceq = _rowsum(eq01, ones)
        frac = (jnp.float32(k) - cgt) / ceq
        wsel = gt01 + frac * eq01
        m_t = jnp.max(tm, axis=1, keepdims=True)
        m_s = jnp.max(sm, axis=1, keepdims=True)
        e_t = jnp.exp(tm - m_t)
        e_s = jnp.exp(sm - m_s)
        total = total + _kl_terms(wsel, e_t, e_s, t, s, m_t, m_s, ones, norm)

    o_ref[0, 0] = jnp.where(pid == 0, total, o_ref[0, 0] + total)


@jax.jit
def kernel(logits, logits_teacher, targets):
    del targets
    out = pl.pallas_call(
        _loss_body,
        grid=(_B // _RB,),
        in_specs=[
            pl.BlockSpec((_RB, _NUM_CLASSES), lambda i: (i, 0)),
            pl.BlockSpec((_RB, _NUM_CLASSES), lambda i: (i, 0)),
        ],
        out_specs=pl.BlockSpec(memory_space=pltpu.SMEM),
        out_shape=jax.ShapeDtypeStruct((1, 1), jnp.float32),
    )(logits, logits_teacher)
    return out[0, 0]
